# Initial kernel scaffold; baseline (speedup 1.0000x reference)
import jax
import jax.numpy as jnp
from jax.experimental import pallas as pl


def kernel(x, emb_table, W1, b1, W2, b2):
    raise NotImplementedError("write your pallas kernel here")


# speedup vs baseline: 19.6688x; 19.6688x over previous
import functools

import jax
import jax.numpy as jnp
from jax import lax
from jax.experimental import pallas as pl
from jax.experimental.pallas import tpu as pltpu
from jax.experimental.pallas import tpu_sc as plsc

VOCAB_P1 = 1000001
EMB = 32
HID = 128
B = 16384
L = 200

ROWS_BLK = 8192
N_BLKS = (VOCAB_P1 + ROWS_BLK - 1) // ROWS_BLK
N_PAD = N_BLKS * ROWS_BLK

NC = 2
NS = 16
NW = NC * NS
NG = B // 128
GPW = NG // NW


def _tw_body(tab_ref, w1_ref, b1_ref, w2_ref, b2_ref, tw_ref, cb_ref):
    w = (w1_ref[...] @ w2_ref[...])[:, 0]
    tb = tab_ref[...]
    tw_ref[...] = jnp.sum(tb * w[None, :], axis=1).reshape(ROWS_BLK // 128, 128)
    c = jnp.sum(b1_ref[...] * w2_ref[:, 0]) + b2_ref[0]
    cb_ref[...] = jnp.full((8, 128), c, jnp.float32)


def _compute_tw(emb_table, w1, b1, w2, b2):
    return pl.pallas_call(
        _tw_body,
        grid=(N_BLKS,),
        in_specs=[
            pl.BlockSpec((ROWS_BLK, EMB), lambda i: (i, 0)),
            pl.BlockSpec((EMB, HID), lambda i: (0, 0)),
            pl.BlockSpec((HID,), lambda i: (0,)),
            pl.BlockSpec((HID, 1), lambda i: (0, 0)),
            pl.BlockSpec((1,), lambda i: (0,)),
        ],
        out_specs=[
            pl.BlockSpec((ROWS_BLK // 128, 128), lambda i: (i, 0)),
            pl.BlockSpec((8, 128), lambda i: (0, 0)),
        ],
        out_shape=[
            jax.ShapeDtypeStruct((N_PAD // 128, 128), jnp.float32),
            jax.ShapeDtypeStruct((8, 128), jnp.float32),
        ],
    )(emb_table, w1, b1, w2, b2)


@functools.cache
def _make_sc_fnn():
    mesh = plsc.VectorSubcoreMesh(core_axis_name="c", subcore_axis_name="s")

    @functools.partial(
        pl.kernel,
        out_type=jax.ShapeDtypeStruct((B,), jnp.float32),
        mesh=mesh,
        scratch_types=[
            pltpu.VMEM((L * 128,), jnp.int32),
            pltpu.VMEM((L * 128,), jnp.float32),
            pltpu.VMEM((128,), jnp.float32),
            pltpu.VMEM((16,), jnp.float32),
            pltpu.VMEM_SHARED((N_PAD,), jnp.float32),
            pltpu.SemaphoreType.DMA,
        ],
    )
    def _sc_fnn(twf_hbm, xt_hbm, cb_hbm, out_hbm, idx_v, vals_v, out_v, c_v,
                tw_sh, sem):
        cid = lax.axis_index("c")
        sid = lax.axis_index("s")
        wid = sid * NC + cid

        @pl.when(sid == 0)
        def _():
            pltpu.sync_copy(twf_hbm, tw_sh)

        pltpu.sync_copy(cb_hbm.at[pl.ds(0, 16)], c_v)
        plsc.subcore_barrier()
        cvec = c_v[...]

        zeros = jnp.zeros((16,), jnp.float32)
        for t in range(GPW):
            g128 = wid * GPW + t
            pltpu.sync_copy(xt_hbm.at[g128], idx_v)
            pltpu.async_copy(tw_sh.at[idx_v], vals_v, sem).wait()

            def body(l, accs):
                base = l * 128
                return tuple(
                    accs[g] + vals_v[pl.ds(base + g * 16, 16)]
                    for g in range(8)
                )

            accs = lax.fori_loop(0, L, body, (zeros,) * 8)
            for g in range(8):
                z = accs[g] * (1.0 / L) + cvec
                out_v[pl.ds(g * 16, 16)] = 1.0 / (1.0 + jnp.exp(-z))
            pltpu.sync_copy(out_v, out_hbm.at[pl.ds(g128 * 128, 128)])

    return _sc_fnn


def kernel(x, emb_table, W1, b1, W2, b2):
    tw, cb = _compute_tw(emb_table, W1, b1, W2, b2)
    twf = tw.reshape(N_PAD)
    cbv = cb[0]
    xt = x.astype(jnp.int32).reshape(NG, 128, L).transpose(0, 2, 1)
    xt = xt.reshape(NG, L * 128)
    out = _make_sc_fnn()(twf, xt, cbv)
    return out.reshape(B, 1)


# speedup vs baseline: 21.9886x; 1.1179x over previous
import functools

import jax
import jax.numpy as jnp
from jax import lax
from jax.experimental import pallas as pl
from jax.experimental.pallas import tpu as pltpu
from jax.experimental.pallas import tpu_sc as plsc

VOCAB_P1 = 1000001
EMB = 32
HID = 128
B = 16384
L = 200

ROWS_BLK = 32768
N_BLKS = (VOCAB_P1 + ROWS_BLK - 1) // ROWS_BLK
N_PAD = N_BLKS * ROWS_BLK

NC = 2
NS = 16
NW = NC * NS
NG = B // 128
GPW = NG // NW


def _tw_body(tab_ref, w1_ref, b1_ref, w2_ref, b2_ref, tw_ref, cb_ref):
    w = (w1_ref[...] @ w2_ref[...])[:, 0]
    tb = tab_ref[...]
    tw_ref[...] = jnp.sum(tb * w[None, :], axis=1).reshape(ROWS_BLK // 128, 128)
    c = jnp.sum(b1_ref[...] * w2_ref[:, 0]) + b2_ref[0]
    cb_ref[...] = jnp.full((8, 128), c, jnp.float32)


def _compute_tw(emb_table, w1, b1, w2, b2):
    return pl.pallas_call(
        _tw_body,
        grid=(N_BLKS,),
        in_specs=[
            pl.BlockSpec((ROWS_BLK, EMB), lambda i: (i, 0)),
            pl.BlockSpec((EMB, HID), lambda i: (0, 0)),
            pl.BlockSpec((HID,), lambda i: (0,)),
            pl.BlockSpec((HID, 1), lambda i: (0, 0)),
            pl.BlockSpec((1,), lambda i: (0,)),
        ],
        out_specs=[
            pl.BlockSpec((ROWS_BLK // 128, 128), lambda i: (i, 0)),
            pl.BlockSpec((8, 128), lambda i: (0, 0)),
        ],
        out_shape=[
            jax.ShapeDtypeStruct((N_PAD // 128, 128), jnp.float32),
            jax.ShapeDtypeStruct((8, 128), jnp.float32),
        ],
    )(emb_table, w1, b1, w2, b2)


@functools.cache
def _make_sc_fnn():
    mesh = plsc.VectorSubcoreMesh(core_axis_name="c", subcore_axis_name="s")

    @functools.partial(
        pl.kernel,
        out_type=jax.ShapeDtypeStruct((B,), jnp.float32),
        mesh=mesh,
        scratch_types=[
            pltpu.VMEM((L * 128,), jnp.int32),
            pltpu.VMEM((L * 128,), jnp.float32),
            pltpu.VMEM((128,), jnp.float32),
            pltpu.VMEM((16,), jnp.float32),
            pltpu.VMEM_SHARED((N_PAD,), jnp.float32),
            pltpu.SemaphoreType.DMA,
        ],
    )
    def _sc_fnn(twf_hbm, xt_hbm, cb_hbm, out_hbm, idx_v, vals_v, out_v, c_v,
                tw_sh, sem):
        cid = lax.axis_index("c")
        sid = lax.axis_index("s")
        wid = sid * NC + cid

        @pl.when(sid == 0)
        def _():
            pltpu.sync_copy(twf_hbm, tw_sh)

        pltpu.sync_copy(cb_hbm.at[pl.ds(0, 16)], c_v)
        plsc.subcore_barrier()
        cvec = c_v[...]

        zeros = jnp.zeros((16,), jnp.float32)
        for t in range(GPW):
            g128 = wid * GPW + t
            pltpu.sync_copy(xt_hbm.at[g128], idx_v)
            pltpu.async_copy(tw_sh.at[idx_v], vals_v, sem).wait()

            def body(l, accs):
                base = l * 128
                return tuple(
                    accs[g] + vals_v[pl.ds(base + g * 16, 16)]
                    for g in range(8)
                )

            accs = lax.fori_loop(0, L, body, (zeros,) * 8)
            for g in range(8):
                z = accs[g] * (1.0 / L) + cvec
                out_v[pl.ds(g * 16, 16)] = 1.0 / (1.0 + jnp.exp(-z))
            pltpu.sync_copy(out_v, out_hbm.at[pl.ds(g128 * 128, 128)])

    return _sc_fnn


def kernel(x, emb_table, W1, b1, W2, b2):
    tw, cb = _compute_tw(emb_table, W1, b1, W2, b2)
    twf = tw.reshape(N_PAD)
    cbv = cb[0]
    xt = x.astype(jnp.int32).reshape(NG, 128, L).transpose(0, 2, 1)
    xt = xt.reshape(NG, L * 128)
    out = _make_sc_fnn()(twf, xt, cbv)
    return out.reshape(B, 1)
